# baseline (speedup 1.0000x reference)
import functools

import jax
import jax.numpy as jnp
from jax import lax
from jax.experimental import pallas as pl
from jax.experimental.pallas import tpu as pltpu
from jax.experimental.pallas import tpu_sc as plsc

VOCAB = 1000
DIM = 128
B = 1024
L = 50
TOKENS = B * L
NW = 32
CHUNK = 80
LB = 5
NSEG = DIM // 16
PAIR_UNROLL = 8
STAGES = ((50, 2, 10),)


def _gather_rows(table, idx, tok_off, tokens, cps, nsuper):
    bpw = tokens // NW
    sup = CHUNK * cps
    npair = sup // 2
    assert bpw == sup * nsuper and nsuper >= 2 and npair % PAIR_UNROLL == 0
    mesh = plsc.VectorSubcoreMesh(core_axis_name="c", subcore_axis_name="s")

    @functools.partial(
        pl.kernel,
        mesh=mesh,
        out_type=jax.ShapeDtypeStruct((tokens // 2, DIM), jnp.float32),
        scratch_types=[
            pltpu.VMEM((bpw,), jnp.int32),
            pltpu.VMEM((sup, DIM), jnp.float32),
            pltpu.VMEM((sup, DIM), jnp.float32),
            pltpu.VMEM((npair, DIM), jnp.float32),
            pltpu.VMEM((npair, DIM), jnp.float32),
            pltpu.VMEM_SHARED((VOCAB, DIM), jnp.float32),
            pltpu.SemaphoreType.DMA,
            pltpu.SemaphoreType.DMA,
            pltpu.SemaphoreType.DMA,
            pltpu.SemaphoreType.DMA,
        ],
        compiler_params=pltpu.CompilerParams(needs_layout_passes=False),
    )
    def k(table_hbm, idx_hbm, out_hbm, idx_v, gb0, gb1, pb0, pb1, tbl_s,
          sg0, sg1, sw0, sw1):
        gbufs = (gb0, gb1)
        pbufs = (pb0, pb1)
        sgs = (sg0, sg1)
        sws = (sw0, sw1)
        sid = lax.axis_index("s")
        wid = sid * 2 + lax.axis_index("c")
        base = wid * bpw

        @pl.when(sid == 0)
        def _():
            pltpu.sync_copy(table_hbm, tbl_s)

        pltpu.sync_copy(idx_hbm.at[pl.ds(tok_off + base, bpw)], idx_v)
        plsc.subcore_barrier()

        def fire_gathers(s, q):
            for c in range(cps):
                pltpu.async_copy(
                    tbl_s.at[idx_v.at[pl.ds(s * sup + c * CHUNK, CHUNK)]],
                    gbufs[q].at[pl.ds(c * CHUNK, CHUNK)], sgs[q],
                )

        def drain_gathers(s, q):
            for c in range(cps):
                pltpu.make_async_copy(
                    tbl_s.at[idx_v.at[pl.ds(s * sup + c * CHUNK, CHUNK)]],
                    gbufs[q].at[pl.ds(c * CHUNK, CHUNK)], sgs[q],
                ).wait()

        def pack_pairs(q):
            gq = gbufs[q]
            pq = pbufs[q]

            def body(pr, carry):
                gbase = pl.multiple_of(pr * 2 * PAIR_UNROLL, 8)
                pbase = pl.multiple_of(pr * PAIR_UNROLL, 8)
                for u in range(PAIR_UNROLL):
                    for seg in range(NSEG):
                        a = gq[gbase + 2 * u, pl.ds(seg * 16, 16)]
                        b = gq[gbase + 2 * u + 1, pl.ds(seg * 16, 16)]
                        pk = plsc.pack(a, b, format=plsc.PackFormat.INTERLEAVED)
                        pq[pbase + u, pl.ds(seg * 16, 16)] = plsc.bitcast(
                            pk, jnp.float32)
                return carry

            lax.fori_loop(0, npair // PAIR_UNROLL, body, 0)

        obase = pl.multiple_of(wid * (bpw // 2), 8)

        def write(s, q):
            pltpu.async_copy(
                pbufs[q],
                out_hbm.at[pl.ds(obase + s * npair, npair)], sws[q]
            )

        def wait_write(s, q):
            pltpu.make_async_copy(
                pbufs[q],
                out_hbm.at[pl.ds(obase + s * npair, npair)], sws[q]
            ).wait()

        fire_gathers(0, 0)
        for s in range(nsuper):
            q = s % 2
            drain_gathers(s, q)
            if s + 1 < nsuper:
                fire_gathers(s + 1, 1 - q)
            if s >= 2:
                wait_write(s - 2, q)
            pack_pairs(q)
            write(s, q)
        wait_write(nsuper - 2, (nsuper - 2) % 2)
        wait_write(nsuper - 1, (nsuper - 1) % 2)

    return k(table, idx)


def _head_matmul(h3, w, b2, nl, l_off, out_prev=None):
    def mm(h_ref, w_ref, b_ref, *refs):
        o_ref = refs[-1]
        wv = w_ref[...].astype(jnp.bfloat16)
        bv = b_ref[...]
        for i in range(LB):
            hl = pltpu.bitcast(h_ref[i], jnp.bfloat16).reshape(B, DIM)
            acc = lax.dot_general(
                wv, hl,
                dimension_numbers=(((1,), (1,)), ((), ())),
                preferred_element_type=jnp.float32,
            )
            o_ref[i] = acc + bv

    off = l_off // LB
    in_specs = [
        pl.BlockSpec((LB, B // 2, DIM), lambda l: (l, 0, 0)),
        pl.BlockSpec((VOCAB, DIM), lambda l: (0, 0)),
        pl.BlockSpec((VOCAB, 1), lambda l: (0, 0)),
    ]
    args = [h3, w, b2]
    kwargs = {}
    if out_prev is not None:
        in_specs.append(pl.BlockSpec(memory_space=pl.ANY))
        args.append(out_prev)
        kwargs = dict(input_output_aliases={3: 0})
    return pl.pallas_call(
        mm,
        grid=(nl // LB,),
        in_specs=in_specs,
        out_specs=pl.BlockSpec((LB, VOCAB, B), lambda l, off=off: (l + off, 0, 0)),
        out_shape=jax.ShapeDtypeStruct((L, VOCAB, B), jnp.float32),
        **kwargs,
    )(*args)


def kernel(input_ids, embed_table, head_w, head_b):
    idx = input_ids.astype(jnp.int32).T.reshape(TOKENS)
    b2 = head_b.reshape(VOCAB, 1)
    hs = []
    l_off = 0
    for nl, cps, nsuper in STAGES:
        hs.append(_gather_rows(embed_table, idx, l_off * B, nl * B,
                               cps=cps, nsuper=nsuper))
        l_off += nl
    out = None
    l_off = 0
    for (nl, _, _), h in zip(STAGES, hs):
        out = _head_matmul(h.reshape(nl, B // 2, DIM), head_w, b2, nl, l_off,
                           out_prev=out)
        l_off += nl
    return jnp.transpose(out, (2, 0, 1))


# speedup vs baseline: 1.1962x; 1.1962x over previous
import functools

import jax
import jax.numpy as jnp
from jax import lax
from jax.experimental import pallas as pl
from jax.experimental.pallas import tpu as pltpu
from jax.experimental.pallas import tpu_sc as plsc

VOCAB = 1000
DIM = 128
B = 1024
L = 50
TOKENS = B * L
NW = 32
CHUNK = 80
LB = 5
STAGES = ((50, 5, 4),)


def _gather_rows(table, idx, tok_off, tokens, cps, nsuper):
    bpw = tokens // NW
    sup = CHUNK * cps
    assert bpw == sup * nsuper and nsuper >= 2
    mesh = plsc.VectorSubcoreMesh(core_axis_name="c", subcore_axis_name="s")

    @functools.partial(
        pl.kernel,
        mesh=mesh,
        out_type=jax.ShapeDtypeStruct((tokens, DIM), jnp.float32),
        scratch_types=[
            pltpu.VMEM((bpw,), jnp.int32),
            pltpu.VMEM((sup, DIM), jnp.float32),
            pltpu.VMEM((sup, DIM), jnp.float32),
            pltpu.VMEM_SHARED((VOCAB, DIM), jnp.float32),
            pltpu.SemaphoreType.DMA,
            pltpu.SemaphoreType.DMA,
            pltpu.SemaphoreType.DMA,
            pltpu.SemaphoreType.DMA,
        ],
    )
    def k(table_hbm, idx_hbm, out_hbm, idx_v, buf0, buf1, tbl_s,
          sg0, sg1, sw0, sw1):
        bufs = (buf0, buf1)
        sgs = (sg0, sg1)
        sws = (sw0, sw1)
        sid = lax.axis_index("s")
        wid = sid * 2 + lax.axis_index("c")
        base = wid * bpw

        @pl.when(sid == 0)
        def _():
            pltpu.sync_copy(table_hbm, tbl_s)

        pltpu.sync_copy(idx_hbm.at[pl.ds(tok_off + base, bpw)], idx_v)
        plsc.subcore_barrier()

        def fire_gathers(s, q):
            for c in range(cps):
                pltpu.async_copy(
                    tbl_s.at[idx_v.at[pl.ds(s * sup + c * CHUNK, CHUNK)]],
                    bufs[q].at[pl.ds(c * CHUNK, CHUNK)], sgs[q],
                )

        def drain_gathers(s, q):
            for c in range(cps):
                pltpu.make_async_copy(
                    tbl_s.at[idx_v.at[pl.ds(s * sup + c * CHUNK, CHUNK)]],
                    bufs[q].at[pl.ds(c * CHUNK, CHUNK)], sgs[q],
                ).wait()

        def write(s, q):
            pltpu.async_copy(
                bufs[q], out_hbm.at[pl.ds(base + s * sup, sup)], sws[q]
            )

        def wait_write(s, q):
            pltpu.make_async_copy(
                bufs[q], out_hbm.at[pl.ds(base + s * sup, sup)], sws[q]
            ).wait()

        fire_gathers(0, 0)
        for s in range(nsuper):
            q = s % 2
            drain_gathers(s, q)
            if s + 1 < nsuper:
                if s >= 1:
                    wait_write(s - 1, 1 - q)
                fire_gathers(s + 1, 1 - q)
            write(s, q)
        wait_write(nsuper - 2, (nsuper - 2) % 2)
        wait_write(nsuper - 1, (nsuper - 1) % 2)

    return k(table, idx)


def _head_matmul(h3, w, b2, nl, l_off, out_prev=None):
    def mm(h_ref, w_ref, b_ref, *refs):
        o_ref = refs[-1]
        wv = w_ref[...].astype(jnp.bfloat16)
        bv = b_ref[...]
        for i in range(LB):
            hl = h_ref[i].reshape(B, DIM).astype(jnp.bfloat16)
            acc = lax.dot_general(
                wv, hl,
                dimension_numbers=(((1,), (1,)), ((), ())),
                preferred_element_type=jnp.float32,
            )
            o_ref[i] = acc + bv

    off = l_off // LB
    in_specs = [
        pl.BlockSpec((LB, B, DIM), lambda l: (l, 0, 0)),
        pl.BlockSpec((VOCAB, DIM), lambda l: (0, 0)),
        pl.BlockSpec((VOCAB, 1), lambda l: (0, 0)),
    ]
    args = [h3, w, b2]
    kwargs = {}
    if out_prev is not None:
        in_specs.append(pl.BlockSpec(memory_space=pl.ANY))
        args.append(out_prev)
        kwargs = dict(input_output_aliases={3: 0})
    return pl.pallas_call(
        mm,
        grid=(nl // LB,),
        in_specs=in_specs,
        out_specs=pl.BlockSpec((LB, VOCAB, B), lambda l, off=off: (l + off, 0, 0)),
        out_shape=jax.ShapeDtypeStruct((L, VOCAB, B), jnp.float32),
        **kwargs,
    )(*args)


def kernel(input_ids, embed_table, head_w, head_b):
    idx = input_ids.astype(jnp.int32).T.reshape(TOKENS)
    b2 = head_b.reshape(VOCAB, 1)
    hs = []
    l_off = 0
    for nl, cps, nsuper in STAGES:
        hs.append(_gather_rows(embed_table, idx, l_off * B, nl * B,
                               cps=cps, nsuper=nsuper))
        l_off += nl
    out = None
    l_off = 0
    for (nl, _, _), h in zip(STAGES, hs):
        out = _head_matmul(h.reshape(nl, B, DIM), head_w, b2, nl, l_off,
                           out_prev=out)
        l_off += nl
    return jnp.transpose(out, (2, 0, 1))
